# baseline (speedup 1.0000x reference)
import jax
import jax.numpy as jnp
from jax.experimental import pallas as pl


def _posenc_kernel(x_ref, row_ref, col_ref, o_ref):
    j = pl.program_id(1)
    H = x_ref.shape[2]
    W = x_ref.shape[3]

    @pl.when(j == 0)
    def _():
        row_t = row_ref[:H, :].T
        o_ref[0] = x_ref[0] + row_t[:, :, None]

    @pl.when(j == 1)
    def _():
        col_t = col_ref[:W, :].T
        o_ref[0] = x_ref[0] + col_t[:, None, :]


def kernel(x, row_embed, col_embed):
    b, c, h, w = x.shape
    ch = c // 2
    grid = (b, 2)
    return pl.pallas_call(
        _posenc_kernel,
        grid=grid,
        in_specs=[
            pl.BlockSpec((1, ch, h, w), lambda i, j: (i, j, 0, 0)),
            pl.BlockSpec(row_embed.shape, lambda i, j: (0, 0)),
            pl.BlockSpec(col_embed.shape, lambda i, j: (0, 0)),
        ],
        out_specs=pl.BlockSpec((1, ch, h, w), lambda i, j: (i, j, 0, 0)),
        out_shape=jax.ShapeDtypeStruct((b, c, h, w), x.dtype),
    )(x, row_embed, col_embed)


# speedup vs baseline: 1.1270x; 1.1270x over previous
import jax
import jax.numpy as jnp
from jax.experimental import pallas as pl
from jax.experimental.pallas import tpu as pltpu


def _posenc_kernel(x_ref, row_ref, col_ref, o_ref, pos_ref, *, H, W, CH, BS):
    i = pl.program_id(0)
    j = pl.program_id(1)
    HW = H * W

    @pl.when((i == 0) & (j == 0))
    def _build_pos():
        lane = jax.lax.broadcasted_iota(jnp.int32, (H, HW), 1)
        sub = jax.lax.broadcasted_iota(jnp.int32, (H, HW), 0)
        E = (lane // W == sub).astype(jnp.float32)
        F = (lane % W == sub).astype(jnp.float32)
        row_t = row_ref[:H, :].T
        col_t = col_ref[:W, :].T
        pos_ref[:CH, :] = jax.lax.dot(
            row_t, E, precision=jax.lax.Precision.HIGHEST,
            preferred_element_type=jnp.float32)
        pos_ref[CH:, :] = jax.lax.dot(
            col_t, F, precision=jax.lax.Precision.HIGHEST,
            preferred_element_type=jnp.float32)

    o_ref[...] = x_ref[...] + pos_ref[pl.ds(j * BS, BS), :]


def kernel(x, row_embed, col_embed):
    b, c, h, w = x.shape
    ch = c // 2
    hw = h * w
    x2 = x.reshape(b * c, hw)
    BS = 96
    nj = c // BS
    import functools
    body = functools.partial(_posenc_kernel, H=h, W=w, CH=ch, BS=BS)
    out = pl.pallas_call(
        body,
        grid=(b, nj),
        in_specs=[
            pl.BlockSpec((BS, hw), lambda i, j: (i * (c // BS) + j, 0)),
            pl.BlockSpec(row_embed.shape, lambda i, j: (0, 0)),
            pl.BlockSpec(col_embed.shape, lambda i, j: (0, 0)),
        ],
        out_specs=pl.BlockSpec((BS, hw), lambda i, j: (i * (c // BS) + j, 0)),
        out_shape=jax.ShapeDtypeStruct((b * c, hw), x.dtype),
        scratch_shapes=[pltpu.VMEM((c, hw), jnp.float32)],
    )(x2, row_embed, col_embed)
    return out.reshape(b, c, h, w)


# speedup vs baseline: 8.8908x; 7.8889x over previous
import functools

import jax
import jax.numpy as jnp
from jax.experimental import pallas as pl
from jax.experimental.pallas import tpu as pltpu


def _posenc_kernel(x_ref, row_ref, col_ref, o_ref, pos_ref, *, H, W, CH):
    i = pl.program_id(0)
    HW = H * W

    @pl.when(i == 0)
    def _build_pos():
        r = jax.lax.broadcasted_iota(jnp.int32, (HW, H), 0)
        k = jax.lax.broadcasted_iota(jnp.int32, (HW, H), 1)
        Eh = (r // W == k).astype(jnp.float32)
        Ew = (r % W == k).astype(jnp.float32)
        pos_ref[:, :CH] = jax.lax.dot(
            Eh, row_ref[:H, :], precision=jax.lax.Precision.HIGHEST,
            preferred_element_type=jnp.float32)
        pos_ref[:, CH:] = jax.lax.dot(
            Ew, col_ref[:W, :], precision=jax.lax.Precision.HIGHEST,
            preferred_element_type=jnp.float32)

    o_ref[...] = x_ref[...] + pos_ref[...]


def kernel(x, row_embed, col_embed):
    b, c, h, w = x.shape
    ch = c // 2
    hw = h * w
    xt = jnp.transpose(x, (0, 2, 3, 1)).reshape(b * hw, c)
    body = functools.partial(_posenc_kernel, H=h, W=w, CH=ch)
    out = pl.pallas_call(
        body,
        grid=(b,),
        in_specs=[
            pl.BlockSpec((hw, c), lambda i: (i, 0)),
            pl.BlockSpec(row_embed.shape, lambda i: (0, 0)),
            pl.BlockSpec(col_embed.shape, lambda i: (0, 0)),
        ],
        out_specs=pl.BlockSpec((hw, c), lambda i: (i, 0)),
        out_shape=jax.ShapeDtypeStruct((b * hw, c), x.dtype),
        scratch_shapes=[pltpu.VMEM((hw, c), jnp.float32)],
    )(xt, row_embed, col_embed)
    return out.reshape(b, h, w, c).transpose(0, 3, 1, 2)
